# baseline (speedup 1.0000x reference)
import functools

import jax
import jax.numpy as jnp
from jax import lax
from jax.experimental import pallas as pl
from jax.experimental.pallas import tpu as pltpu
from jax.experimental.pallas import tpu_sc as plsc

N_NODES = 50000
D = 128
NEIGH = 7
NW = 32
B = 112
NBLK = 14
PER_W = B * NBLK
NPAD = NW * PER_W
MT = 400


def _selu(x):
    scale = 1.0507009873554805
    alpha = 1.6732632423543772
    return scale * jnp.where(x > 0, x, alpha * (jnp.exp(x) - 1.0))


def _mm7_body(x_ref, w_ref, o_ref):
    x = x_ref[...].astype(jnp.bfloat16)
    for k in range(NEIGH):
        o_ref[k] = jnp.dot(x, w_ref[k], preferred_element_type=jnp.float32)


def _mm7(x, wstack):
    nb = N_NODES // MT
    return pl.pallas_call(
        _mm7_body,
        grid=(nb,),
        in_specs=[
            pl.BlockSpec((MT, D), lambda i: (i, 0)),
            pl.BlockSpec((NEIGH, D, D), lambda i: (0, 0, 0)),
        ],
        out_specs=pl.BlockSpec((NEIGH, MT, D), lambda i: (0, i, 0)),
        out_shape=jax.ShapeDtypeStruct((NEIGH, NPAD, D), jnp.float32),
    )(x, wstack)


def _mid_body(h_ref, b1_ref, w2_ref, b2_ref, w3_ref, o_ref):
    x1 = _selu(h_ref[...] + b1_ref[...]).astype(jnp.bfloat16)
    t = _selu(jnp.dot(x1, w2_ref[...], preferred_element_type=jnp.float32)
              + b2_ref[...]).astype(jnp.bfloat16)
    for k in range(NEIGH):
        o_ref[k] = jnp.dot(t, w3_ref[k], preferred_element_type=jnp.float32)


def _mid(h, b1, w2, b2, w3stack):
    nb = N_NODES // MT
    return pl.pallas_call(
        _mid_body,
        grid=(nb,),
        in_specs=[
            pl.BlockSpec((MT, D), lambda i: (i, 0)),
            pl.BlockSpec((1, D), lambda i: (0, 0)),
            pl.BlockSpec((D, D), lambda i: (0, 0)),
            pl.BlockSpec((1, D), lambda i: (0, 0)),
            pl.BlockSpec((NEIGH, D, D), lambda i: (0, 0, 0)),
        ],
        out_specs=pl.BlockSpec((NEIGH, MT, D), lambda i: (0, i, 0)),
        out_shape=jax.ShapeDtypeStruct((NEIGH, NPAD, D), jnp.float32),
    )(h, b1, w2, b2, w3stack)


def _out_body(h_ref, b3_ref, w4_ref, b4_ref, w5_ref, b5_ref, o_ref):
    x2 = _selu(h_ref[...] + b3_ref[...]).astype(jnp.bfloat16)
    t = _selu(jnp.dot(x2, w4_ref[...], preferred_element_type=jnp.float32)
              + b4_ref[...]).astype(jnp.bfloat16)
    z = jnp.dot(t, w5_ref[...], preferred_element_type=jnp.float32) + b5_ref[...]
    o_ref[...] = 1.0 / (1.0 + jnp.exp(-z))


def _out_stage(h, b3, w4, b4, w5two, b5two):
    nb = N_NODES // MT
    return pl.pallas_call(
        _out_body,
        grid=(nb,),
        in_specs=[
            pl.BlockSpec((MT, D), lambda i: (i, 0)),
            pl.BlockSpec((1, D), lambda i: (0, 0)),
            pl.BlockSpec((D, D), lambda i: (0, 0)),
            pl.BlockSpec((1, D), lambda i: (0, 0)),
            pl.BlockSpec((D, 8), lambda i: (0, 0)),
            pl.BlockSpec((1, 8), lambda i: (0, 0)),
        ],
        out_specs=pl.BlockSpec((MT, 8), lambda i: (i, 0)),
        out_shape=jax.ShapeDtypeStruct((N_NODES, 8), jnp.float32),
    )(h, b3, w4, b4, w5two, b5two)


def _sc_mesh():
    return plsc.VectorSubcoreMesh(core_axis_name="c", subcore_axis_name="s")


def _accum_pair(acc, ga, gb):
    @pl.loop(0, B)
    def _row(r):
        for j in range(D // 16):
            sl = (pl.ds(r, 1), pl.ds(j * 16, 16))
            plsc.addupdate(acc.at[sl], ga[sl] + gb[sl])


def _gather_sum(table, idx_blocks):
    @functools.partial(
        pl.kernel,
        out_type=jax.ShapeDtypeStruct((NPAD, D), jnp.float32),
        mesh=_sc_mesh(),
        scratch_types=[
            pltpu.VMEM((NEIGH, B), jnp.int32),
            pltpu.VMEM((B, D), jnp.float32),
            pltpu.VMEM((B, D), jnp.float32),
            pltpu.VMEM((B, D), jnp.float32),
            pltpu.VMEM((B, D), jnp.float32),
            pltpu.VMEM((B, D), jnp.float32),
            pltpu.SemaphoreType.DMA,
            pltpu.SemaphoreType.DMA,
            pltpu.SemaphoreType.DMA,
            pltpu.SemaphoreType.DMA,
            pltpu.SemaphoreType.DMA,
        ],
    )
    def k(table_hbm, idx_hbm, out_hbm, idx_v, acc, g0, g1, g2, g3,
          sa, s0, s1, s2, s3):
        cid = lax.axis_index("c")
        sid = lax.axis_index("s")
        wid = sid * 2 + cid

        @pl.loop(0, NBLK)
        def _blk(blk):
            base = wid * PER_W + blk * B
            pltpu.sync_copy(idx_hbm.at[wid * NBLK + blk], idx_v)
            ca = pltpu.async_copy(table_hbm.at[idx_v.at[0]], acc, sa)
            c0 = pltpu.async_copy(table_hbm.at[idx_v.at[1]], g0, s0)
            c1 = pltpu.async_copy(table_hbm.at[idx_v.at[2]], g1, s1)
            c2 = pltpu.async_copy(table_hbm.at[idx_v.at[3]], g2, s2)
            c3 = pltpu.async_copy(table_hbm.at[idx_v.at[4]], g3, s3)
            ca.wait()
            c0.wait()
            c1.wait()
            _accum_pair(acc, g0, g1)
            c5 = pltpu.async_copy(table_hbm.at[idx_v.at[5]], g0, s0)
            c6 = pltpu.async_copy(table_hbm.at[idx_v.at[6]], g1, s1)
            c2.wait()
            c3.wait()
            _accum_pair(acc, g2, g3)
            c5.wait()
            c6.wait()
            _accum_pair(acc, g0, g1)
            pltpu.sync_copy(acc, out_hbm.at[pl.ds(base, B)])

    return k(table, idx_blocks)


def kernel(logits, G, W1, b1, W2, b2, W3, b3, W4, b4, W5, b5):
    f32 = jnp.float32
    bf16 = jnp.bfloat16
    x = logits.astype(f32)
    npadrows = NPAD - N_NODES

    Gi = G.astype(jnp.int32)
    gpad = (jnp.arange(npadrows, dtype=jnp.int32)[:, None] * NEIGH
            + jnp.arange(NEIGH, dtype=jnp.int32)[None, :]) % N_NODES
    Gp = jnp.concatenate([Gi, gpad], axis=0)
    idxT = (Gp.T + (jnp.arange(NEIGH, dtype=jnp.int32) * NPAD)[:, None])
    idxT = idxT.astype(jnp.int32)
    idx_blocks = (idxT.reshape(NEIGH, NW, NBLK, B)
                  .transpose(1, 2, 0, 3)
                  .reshape(NW * NBLK, NEIGH, B))

    w1s = W1.astype(bf16).reshape(NEIGH, D, D)
    w3s = W3.astype(bf16).reshape(NEIGH, D, D)
    b1r = b1.astype(f32).reshape(1, D)
    b2r = b2.astype(f32).reshape(1, D)
    b3r = b3.astype(f32).reshape(1, D)
    b4r = b4.astype(f32).reshape(1, D)
    wd = jnp.stack([W5[:, 0] - W5[:, 1], W5[:, 1] - W5[:, 0]], axis=1)
    w5two = jnp.concatenate([wd.astype(f32), jnp.zeros((D, 6), f32)], axis=1)
    w5two = w5two.astype(bf16)
    bd = jnp.stack([b5[0] - b5[1], b5[1] - b5[0]])
    b5two = jnp.concatenate([bd.astype(f32), jnp.zeros((6,), f32)])
    b5two = b5two.reshape(1, 8)

    y1 = _mm7(x, w1s).reshape(NEIGH * NPAD, D)
    h1 = _gather_sum(y1, idx_blocks)
    y2 = _mid(h1, b1r, W2.astype(bf16), b2r, w3s).reshape(NEIGH * NPAD, D)
    h2 = _gather_sum(y2, idx_blocks)
    out = _out_stage(h2, b3r, W4.astype(bf16), b4r, w5two, b5two)
    return out[: N_NODES - 1, :2]


# speedup vs baseline: 1.2147x; 1.2147x over previous
import functools

import jax
import jax.numpy as jnp
from jax import lax
from jax.experimental import pallas as pl
from jax.experimental.pallas import tpu as pltpu
from jax.experimental.pallas import tpu_sc as plsc

N_NODES = 50000
D = 128
NEIGH = 7
NW = 32
B = 112
NBLK = 14
PER_W = B * NBLK
NPAD = NW * PER_W
CHUNKS = 2
NBLK_C = NBLK // CHUNKS
CROWS = NPAD // CHUNKS
MT = 448
CNB = CROWS // MT


def _selu(x):
    scale = 1.0507009873554805
    alpha = 1.6732632423543772
    return scale * jnp.where(x > 0, x, alpha * (jnp.exp(x) - 1.0))


def _mm7_body(x_ref, w_ref, o_ref):
    x = x_ref[...].astype(jnp.bfloat16)
    for k in range(NEIGH):
        o_ref[k] = jnp.dot(x, w_ref[k], preferred_element_type=jnp.float32)


def _mm7(x, wstack):
    nb = N_NODES // MT + 1
    return pl.pallas_call(
        _mm7_body,
        grid=(nb,),
        in_specs=[
            pl.BlockSpec((MT, D), lambda i: (i, 0)),
            pl.BlockSpec((NEIGH, D, D), lambda i: (0, 0, 0)),
        ],
        out_specs=pl.BlockSpec((NEIGH, MT, D), lambda i: (0, i, 0)),
        out_shape=jax.ShapeDtypeStruct((NEIGH, NPAD, D), jnp.float32),
    )(x, wstack)


def _mid_body(h_ref, b1_ref, w2_ref, b2_ref, w3_ref, o_ref):
    x1 = _selu(h_ref[...] + b1_ref[...]).astype(jnp.bfloat16)
    t = _selu(jnp.dot(x1, w2_ref[...], preferred_element_type=jnp.float32)
              + b2_ref[...]).astype(jnp.bfloat16)
    for k in range(NEIGH):
        o_ref[k] = jnp.dot(t, w3_ref[k], preferred_element_type=jnp.float32)


def _mid_body_alias(h_ref, b1_ref, w2_ref, b2_ref, w3_ref, y_ref, o_ref):
    _mid_body(h_ref, b1_ref, w2_ref, b2_ref, w3_ref, o_ref)


def _mid(h_chunk, b1, w2, b2, w3stack, ybuf, chunk):
    off = chunk * CNB
    specs = [
        pl.BlockSpec((MT, D), lambda i: (i, 0)),
        pl.BlockSpec((1, D), lambda i: (0, 0)),
        pl.BlockSpec((D, D), lambda i: (0, 0)),
        pl.BlockSpec((1, D), lambda i: (0, 0)),
        pl.BlockSpec((NEIGH, D, D), lambda i: (0, 0, 0)),
    ]
    args = [h_chunk, b1, w2, b2, w3stack]
    body = _mid_body
    aliases = {}
    if ybuf is not None:
        specs.append(pl.BlockSpec(memory_space=pl.ANY))
        args.append(ybuf)
        body = _mid_body_alias
        aliases = {5: 0}
    return pl.pallas_call(
        body,
        grid=(CNB,),
        in_specs=specs,
        out_specs=pl.BlockSpec((NEIGH, MT, D), lambda i: (0, off + i, 0)),
        out_shape=jax.ShapeDtypeStruct((NEIGH, NPAD, D), jnp.float32),
        input_output_aliases=aliases,
    )(*args)


def _out_body(h_ref, b3_ref, w4_ref, b4_ref, w5_ref, b5_ref, o_ref):
    x2 = _selu(h_ref[...] + b3_ref[...]).astype(jnp.bfloat16)
    t = _selu(jnp.dot(x2, w4_ref[...], preferred_element_type=jnp.float32)
              + b4_ref[...]).astype(jnp.bfloat16)
    z = jnp.dot(t, w5_ref[...], preferred_element_type=jnp.float32) + b5_ref[...]
    o_ref[...] = 1.0 / (1.0 + jnp.exp(-z))


def _out_body_alias(h_ref, b3_ref, w4_ref, b4_ref, w5_ref, b5_ref,
                    obuf_ref, o_ref):
    _out_body(h_ref, b3_ref, w4_ref, b4_ref, w5_ref, b5_ref, o_ref)


def _out_stage(h_chunk, b3, w4, b4, w5two, b5two, obuf, chunk):
    off = chunk * CNB
    specs = [
        pl.BlockSpec((MT, D), lambda i: (i, 0)),
        pl.BlockSpec((1, D), lambda i: (0, 0)),
        pl.BlockSpec((D, D), lambda i: (0, 0)),
        pl.BlockSpec((1, D), lambda i: (0, 0)),
        pl.BlockSpec((D, 2), lambda i: (0, 0)),
        pl.BlockSpec((1, 2), lambda i: (0, 0)),
    ]
    args = [h_chunk, b3, w4, b4, w5two, b5two]
    body = _out_body
    aliases = {}
    if obuf is not None:
        specs.append(pl.BlockSpec(memory_space=pl.ANY))
        args.append(obuf)
        body = _out_body_alias
        aliases = {6: 0}
    return pl.pallas_call(
        body,
        grid=(CNB,),
        in_specs=specs,
        out_specs=pl.BlockSpec((MT, 2), lambda i: (off + i, 0)),
        out_shape=jax.ShapeDtypeStruct((N_NODES - 1, 2), jnp.float32),
        input_output_aliases=aliases,
    )(*args)


def _sc_mesh():
    return plsc.VectorSubcoreMesh(core_axis_name="c", subcore_axis_name="s")


def _accum_pair(acc, ga, gb):
    @pl.loop(0, B)
    def _row(r):
        for j in range(D // 16):
            sl = (pl.ds(r, 1), pl.ds(j * 16, 16))
            plsc.addupdate(acc.at[sl], ga[sl] + gb[sl])


def _gather_sum(table, idx_chunk):
    per_w = PER_W // CHUNKS

    @functools.partial(
        pl.kernel,
        out_type=jax.ShapeDtypeStruct((CROWS, D), jnp.float32),
        mesh=_sc_mesh(),
        scratch_types=[
            pltpu.VMEM((NEIGH, B), jnp.int32),
            pltpu.VMEM((B, D), jnp.float32),
            pltpu.VMEM((B, D), jnp.float32),
            pltpu.VMEM((B, D), jnp.float32),
            pltpu.VMEM((B, D), jnp.float32),
            pltpu.VMEM((B, D), jnp.float32),
            pltpu.SemaphoreType.DMA,
            pltpu.SemaphoreType.DMA,
            pltpu.SemaphoreType.DMA,
            pltpu.SemaphoreType.DMA,
            pltpu.SemaphoreType.DMA,
        ],
    )
    def k(table_hbm, idx_hbm, out_hbm, idx_v, acc, g0, g1, g2, g3,
          sa, s0, s1, s2, s3):
        cid = lax.axis_index("c")
        sid = lax.axis_index("s")
        wid = sid * 2 + cid

        @pl.loop(0, NBLK_C)
        def _blk(blk):
            base = wid * per_w + blk * B
            pltpu.sync_copy(idx_hbm.at[wid * NBLK_C + blk], idx_v)
            ca = pltpu.async_copy(table_hbm.at[idx_v.at[0]], acc, sa)
            c0 = pltpu.async_copy(table_hbm.at[idx_v.at[1]], g0, s0)
            c1 = pltpu.async_copy(table_hbm.at[idx_v.at[2]], g1, s1)
            c2 = pltpu.async_copy(table_hbm.at[idx_v.at[3]], g2, s2)
            c3 = pltpu.async_copy(table_hbm.at[idx_v.at[4]], g3, s3)
            ca.wait()
            c0.wait()
            c1.wait()
            _accum_pair(acc, g0, g1)
            c5 = pltpu.async_copy(table_hbm.at[idx_v.at[5]], g0, s0)
            c6 = pltpu.async_copy(table_hbm.at[idx_v.at[6]], g1, s1)
            c2.wait()
            c3.wait()
            _accum_pair(acc, g2, g3)
            c5.wait()
            c6.wait()
            _accum_pair(acc, g0, g1)
            pltpu.sync_copy(acc, out_hbm.at[pl.ds(base, B)])

    return k(table, idx_chunk)


def kernel(logits, G, W1, b1, W2, b2, W3, b3, W4, b4, W5, b5):
    f32 = jnp.float32
    bf16 = jnp.bfloat16
    x = logits.astype(f32)
    npadrows = NPAD - N_NODES

    Gi = G.astype(jnp.int32)
    gpad = (jnp.arange(npadrows, dtype=jnp.int32)[:, None] * NEIGH
            + jnp.arange(NEIGH, dtype=jnp.int32)[None, :]) % N_NODES
    Gp = jnp.concatenate([Gi, gpad], axis=0)
    idxT = (Gp.T + (jnp.arange(NEIGH, dtype=jnp.int32) * NPAD)[:, None])
    idxT = idxT.astype(jnp.int32)
    idx_chunks = (idxT.reshape(NEIGH, CHUNKS, NW, NBLK_C, B)
                  .transpose(1, 2, 3, 0, 4)
                  .reshape(CHUNKS, NW * NBLK_C, NEIGH, B))

    w1s = W1.astype(bf16).reshape(NEIGH, D, D)
    w3s = W3.astype(bf16).reshape(NEIGH, D, D)
    b1r = b1.astype(f32).reshape(1, D)
    b2r = b2.astype(f32).reshape(1, D)
    b3r = b3.astype(f32).reshape(1, D)
    b4r = b4.astype(f32).reshape(1, D)
    w2c = W2.astype(bf16)
    w4c = W4.astype(bf16)
    wd = jnp.stack([W5[:, 0] - W5[:, 1], W5[:, 1] - W5[:, 0]], axis=1)
    w5two = wd.astype(bf16)
    bd = jnp.stack([b5[0] - b5[1], b5[1] - b5[0]])
    b5two = bd.astype(f32).reshape(1, 2)

    y1 = _mm7(x, w1s).reshape(NEIGH * NPAD, D)

    h1 = [_gather_sum(y1, idx_chunks[c]) for c in range(CHUNKS)]

    ybuf = None
    for c in range(CHUNKS):
        ybuf = _mid(h1[c], b1r, w2c, b2r, w3s, ybuf, c)
    y2 = ybuf.reshape(NEIGH * NPAD, D)

    h2 = [_gather_sum(y2, idx_chunks[c]) for c in range(CHUNKS)]

    obuf = None
    for c in range(CHUNKS):
        obuf = _out_stage(h2[c], b3r, w4c, b4r, w5two, b5two, obuf, c)
    return obuf


# speedup vs baseline: 1.2222x; 1.0062x over previous
import functools

import jax
import jax.numpy as jnp
from jax import lax
from jax.experimental import pallas as pl
from jax.experimental.pallas import tpu as pltpu
from jax.experimental.pallas import tpu_sc as plsc

N_NODES = 50000
D = 128
NEIGH = 7
NW = 32
B = 112
NBLK = 14
PER_W = B * NBLK
NPAD = NW * PER_W
CHUNKS = 2
NBLK_C = NBLK // CHUNKS
CROWS = NPAD // CHUNKS
MT = 448
CNB = CROWS // MT


def _selu(x):
    scale = 1.0507009873554805
    alpha = 1.6732632423543772
    return scale * jnp.where(x > 0, x, alpha * (jnp.exp(x) - 1.0))


def _write_mm7(x, w_ref, o_ref):
    for p in range(3):
        r = jnp.dot(x, w_ref[:, 256 * p: 256 * (p + 1)],
                    preferred_element_type=jnp.float32)
        o_ref[2 * p] = r[:, :D]
        o_ref[2 * p + 1] = r[:, D:]
    o_ref[6] = jnp.dot(x, w_ref[:, 768:896],
                       preferred_element_type=jnp.float32)


def _mm7_body(x_ref, w_ref, o_ref):
    x = x_ref[...].astype(jnp.bfloat16)
    _write_mm7(x, w_ref, o_ref)


def _mm7(x, wstack):
    nb = N_NODES // MT + 1
    return pl.pallas_call(
        _mm7_body,
        grid=(nb,),
        in_specs=[
            pl.BlockSpec((MT, D), lambda i: (i, 0)),
            pl.BlockSpec((D, NEIGH * D), lambda i: (0, 0)),
        ],
        out_specs=pl.BlockSpec((NEIGH, MT, D), lambda i: (0, i, 0)),
        out_shape=jax.ShapeDtypeStruct((NEIGH, NPAD, D), jnp.float32),
    )(x, wstack)


def _mid_body(h_ref, b1_ref, w2_ref, b2_ref, w3_ref, o_ref):
    x1 = _selu(h_ref[...] + b1_ref[...]).astype(jnp.bfloat16)
    t = _selu(jnp.dot(x1, w2_ref[...], preferred_element_type=jnp.float32)
              + b2_ref[...]).astype(jnp.bfloat16)
    _write_mm7(t, w3_ref, o_ref)


def _mid_body_alias(h_ref, b1_ref, w2_ref, b2_ref, w3_ref, y_ref, o_ref):
    _mid_body(h_ref, b1_ref, w2_ref, b2_ref, w3_ref, o_ref)


def _mid(h_chunk, b1, w2, b2, w3stack, ybuf, chunk):
    off = chunk * CNB
    specs = [
        pl.BlockSpec((MT, D), lambda i: (i, 0)),
        pl.BlockSpec((1, D), lambda i: (0, 0)),
        pl.BlockSpec((D, D), lambda i: (0, 0)),
        pl.BlockSpec((1, D), lambda i: (0, 0)),
        pl.BlockSpec((D, NEIGH * D), lambda i: (0, 0)),
    ]
    args = [h_chunk, b1, w2, b2, w3stack]
    body = _mid_body
    aliases = {}
    if ybuf is not None:
        specs.append(pl.BlockSpec(memory_space=pl.ANY))
        args.append(ybuf)
        body = _mid_body_alias
        aliases = {5: 0}
    return pl.pallas_call(
        body,
        grid=(CNB,),
        in_specs=specs,
        out_specs=pl.BlockSpec((NEIGH, MT, D), lambda i: (0, off + i, 0)),
        out_shape=jax.ShapeDtypeStruct((NEIGH, NPAD, D), jnp.float32),
        input_output_aliases=aliases,
    )(*args)


def _out_body(h_ref, b3_ref, w4_ref, b4_ref, w5_ref, b5_ref, o_ref):
    x2 = _selu(h_ref[...] + b3_ref[...]).astype(jnp.bfloat16)
    t = _selu(jnp.dot(x2, w4_ref[...], preferred_element_type=jnp.float32)
              + b4_ref[...]).astype(jnp.bfloat16)
    z = jnp.dot(t, w5_ref[...], preferred_element_type=jnp.float32) + b5_ref[...]
    o_ref[...] = 1.0 / (1.0 + jnp.exp(-z))


def _out_body_alias(h_ref, b3_ref, w4_ref, b4_ref, w5_ref, b5_ref,
                    obuf_ref, o_ref):
    _out_body(h_ref, b3_ref, w4_ref, b4_ref, w5_ref, b5_ref, o_ref)


def _out_stage(h_chunk, b3, w4, b4, w5two, b5two, obuf, chunk):
    off = chunk * CNB
    specs = [
        pl.BlockSpec((MT, D), lambda i: (i, 0)),
        pl.BlockSpec((1, D), lambda i: (0, 0)),
        pl.BlockSpec((D, D), lambda i: (0, 0)),
        pl.BlockSpec((1, D), lambda i: (0, 0)),
        pl.BlockSpec((D, 2), lambda i: (0, 0)),
        pl.BlockSpec((1, 2), lambda i: (0, 0)),
    ]
    args = [h_chunk, b3, w4, b4, w5two, b5two]
    body = _out_body
    aliases = {}
    if obuf is not None:
        specs.append(pl.BlockSpec(memory_space=pl.ANY))
        args.append(obuf)
        body = _out_body_alias
        aliases = {6: 0}
    return pl.pallas_call(
        body,
        grid=(CNB,),
        in_specs=specs,
        out_specs=pl.BlockSpec((MT, 2), lambda i: (off + i, 0)),
        out_shape=jax.ShapeDtypeStruct((N_NODES - 1, 2), jnp.float32),
        input_output_aliases=aliases,
    )(*args)


def _sc_mesh():
    return plsc.VectorSubcoreMesh(core_axis_name="c", subcore_axis_name="s")


def _accum_pair(acc, ga, gb):
    @pl.loop(0, B)
    def _row(r):
        for j in range(D // 16):
            sl = (pl.ds(r, 1), pl.ds(j * 16, 16))
            plsc.addupdate(acc.at[sl], ga[sl] + gb[sl])


def _gather_sum(table, idx_chunk):
    per_w = PER_W // CHUNKS

    @functools.partial(
        pl.kernel,
        out_type=jax.ShapeDtypeStruct((CROWS, D), jnp.float32),
        mesh=_sc_mesh(),
        scratch_types=[
            pltpu.VMEM((NEIGH, B), jnp.int32),
            pltpu.VMEM((B, D), jnp.float32),
            pltpu.VMEM((B, D), jnp.float32),
            pltpu.VMEM((B, D), jnp.float32),
            pltpu.VMEM((B, D), jnp.float32),
            pltpu.VMEM((B, D), jnp.float32),
            pltpu.SemaphoreType.DMA,
            pltpu.SemaphoreType.DMA,
            pltpu.SemaphoreType.DMA,
            pltpu.SemaphoreType.DMA,
            pltpu.SemaphoreType.DMA,
        ],
    )
    def k(table_hbm, idx_hbm, out_hbm, idx_v, acc, g0, g1, g2, g3,
          sa, s0, s1, s2, s3):
        cid = lax.axis_index("c")
        sid = lax.axis_index("s")
        wid = sid * 2 + cid

        @pl.loop(0, NBLK_C)
        def _blk(blk):
            base = wid * per_w + blk * B
            pltpu.sync_copy(idx_hbm.at[wid * NBLK_C + blk], idx_v)
            ca = pltpu.async_copy(table_hbm.at[idx_v.at[0]], acc, sa)
            c0 = pltpu.async_copy(table_hbm.at[idx_v.at[1]], g0, s0)
            c1 = pltpu.async_copy(table_hbm.at[idx_v.at[2]], g1, s1)
            c2 = pltpu.async_copy(table_hbm.at[idx_v.at[3]], g2, s2)
            c3 = pltpu.async_copy(table_hbm.at[idx_v.at[4]], g3, s3)
            ca.wait()
            c0.wait()
            c1.wait()
            _accum_pair(acc, g0, g1)
            c5 = pltpu.async_copy(table_hbm.at[idx_v.at[5]], g0, s0)
            c6 = pltpu.async_copy(table_hbm.at[idx_v.at[6]], g1, s1)
            c2.wait()
            c3.wait()
            _accum_pair(acc, g2, g3)
            c5.wait()
            c6.wait()
            _accum_pair(acc, g0, g1)
            pltpu.sync_copy(acc, out_hbm.at[pl.ds(base, B)])

    return k(table, idx_chunk)


def kernel(logits, G, W1, b1, W2, b2, W3, b3, W4, b4, W5, b5):
    f32 = jnp.float32
    bf16 = jnp.bfloat16
    x = logits.astype(f32)
    npadrows = NPAD - N_NODES

    Gi = G.astype(jnp.int32)
    gpad = (jnp.arange(npadrows, dtype=jnp.int32)[:, None] * NEIGH
            + jnp.arange(NEIGH, dtype=jnp.int32)[None, :]) % N_NODES
    Gp = jnp.concatenate([Gi, gpad], axis=0)
    idxT = (Gp.T + (jnp.arange(NEIGH, dtype=jnp.int32) * NPAD)[:, None])
    idxT = idxT.astype(jnp.int32)
    idx_chunks = (idxT.reshape(NEIGH, CHUNKS, NW, NBLK_C, B)
                  .transpose(1, 2, 3, 0, 4)
                  .reshape(CHUNKS, NW * NBLK_C, NEIGH, B))

    w1s = W1.reshape(NEIGH, D, D).transpose(1, 0, 2).reshape(D, NEIGH * D)
    w1s = w1s.astype(bf16)
    w3s = W3.reshape(NEIGH, D, D).transpose(1, 0, 2).reshape(D, NEIGH * D)
    w3s = w3s.astype(bf16)
    b1r = b1.astype(f32).reshape(1, D)
    b2r = b2.astype(f32).reshape(1, D)
    b3r = b3.astype(f32).reshape(1, D)
    b4r = b4.astype(f32).reshape(1, D)
    w2c = W2.astype(bf16)
    w4c = W4.astype(bf16)
    wd = jnp.stack([W5[:, 0] - W5[:, 1], W5[:, 1] - W5[:, 0]], axis=1)
    w5two = wd.astype(bf16)
    bd = jnp.stack([b5[0] - b5[1], b5[1] - b5[0]])
    b5two = bd.astype(f32).reshape(1, 2)

    y1 = _mm7(x, w1s).reshape(NEIGH * NPAD, D)

    h1 = [_gather_sum(y1, idx_chunks[c]) for c in range(CHUNKS)]

    ybuf = None
    for c in range(CHUNKS):
        ybuf = _mid(h1[c], b1r, w2c, b2r, w3s, ybuf, c)
    y2 = ybuf.reshape(NEIGH * NPAD, D)

    h2 = [_gather_sum(y2, idx_chunks[c]) for c in range(CHUNKS)]

    obuf = None
    for c in range(CHUNKS):
        obuf = _out_stage(h2[c], b3r, w4c, b4r, w5two, b5two, obuf, c)
    return obuf


# speedup vs baseline: 1.3948x; 1.1412x over previous
import functools

import jax
import jax.numpy as jnp
from jax import lax
from jax.experimental import pallas as pl
from jax.experimental.pallas import tpu as pltpu
from jax.experimental.pallas import tpu_sc as plsc

N_NODES = 50000
D = 128
NEIGH = 7
NW = 32
B = 112
NBLK = 14
PER_W = B * NBLK
NPAD = NW * PER_W
CHUNKS = 2
NBLK_C = NBLK // CHUNKS
CROWS = NPAD // CHUNKS
MT = 448
CNB = CROWS // MT


def _selu(x):
    scale = 1.0507009873554805
    alpha = 1.6732632423543772
    return scale * jnp.where(x > 0, x, alpha * (jnp.exp(x) - 1.0))


def _write_mm7(x, w_ref, o_ref):
    for p in range(3):
        r = jnp.dot(x, w_ref[:, 256 * p: 256 * (p + 1)],
                    preferred_element_type=jnp.float32)
        o_ref[2 * p] = r[:, :D]
        o_ref[2 * p + 1] = r[:, D:]
    o_ref[6] = jnp.dot(x, w_ref[:, 768:896],
                       preferred_element_type=jnp.float32)


def _mm7_body(x_ref, w_ref, o_ref):
    x = x_ref[...].astype(jnp.bfloat16)
    _write_mm7(x, w_ref, o_ref)


def _mm7(x, wstack):
    nb = N_NODES // MT + 1
    return pl.pallas_call(
        _mm7_body,
        grid=(nb,),
        in_specs=[
            pl.BlockSpec((MT, D), lambda i: (i, 0)),
            pl.BlockSpec((D, NEIGH * D), lambda i: (0, 0)),
        ],
        out_specs=pl.BlockSpec((NEIGH, MT, D), lambda i: (0, i, 0)),
        out_shape=jax.ShapeDtypeStruct((NEIGH, NPAD, D), jnp.float32),
    )(x, wstack)


def _mid_body(h_ref, b1_ref, w2_ref, b2_ref, w3_ref, o_ref):
    x1 = _selu(h_ref[...] + b1_ref[...]).astype(jnp.bfloat16)
    t = _selu(jnp.dot(x1, w2_ref[...], preferred_element_type=jnp.float32)
              + b2_ref[...]).astype(jnp.bfloat16)
    _write_mm7(t, w3_ref, o_ref)


def _mid_body_alias(h_ref, b1_ref, w2_ref, b2_ref, w3_ref, y_ref, o_ref):
    _mid_body(h_ref, b1_ref, w2_ref, b2_ref, w3_ref, o_ref)


def _mid(h_chunk, b1, w2, b2, w3stack, ybuf, chunk):
    off = chunk * CNB
    specs = [
        pl.BlockSpec((MT, D), lambda i: (i, 0)),
        pl.BlockSpec((1, D), lambda i: (0, 0)),
        pl.BlockSpec((D, D), lambda i: (0, 0)),
        pl.BlockSpec((1, D), lambda i: (0, 0)),
        pl.BlockSpec((D, NEIGH * D), lambda i: (0, 0)),
    ]
    args = [h_chunk, b1, w2, b2, w3stack]
    body = _mid_body
    aliases = {}
    if ybuf is not None:
        specs.append(pl.BlockSpec(memory_space=pl.ANY))
        args.append(ybuf)
        body = _mid_body_alias
        aliases = {5: 0}
    return pl.pallas_call(
        body,
        grid=(CNB,),
        in_specs=specs,
        out_specs=pl.BlockSpec((NEIGH, MT, D), lambda i: (0, off + i, 0)),
        out_shape=jax.ShapeDtypeStruct((NEIGH, NPAD, D), jnp.float32),
        input_output_aliases=aliases,
    )(*args)


def _out_body(h_ref, b3_ref, w4_ref, b4_ref, w5_ref, b5_ref, o_ref):
    x2 = _selu(h_ref[...] + b3_ref[...]).astype(jnp.bfloat16)
    t = _selu(jnp.dot(x2, w4_ref[...], preferred_element_type=jnp.float32)
              + b4_ref[...]).astype(jnp.bfloat16)
    z = jnp.dot(t, w5_ref[...], preferred_element_type=jnp.float32) + b5_ref[...]
    o_ref[...] = 1.0 / (1.0 + jnp.exp(-z))


def _out_body_alias(h_ref, b3_ref, w4_ref, b4_ref, w5_ref, b5_ref,
                    obuf_ref, o_ref):
    _out_body(h_ref, b3_ref, w4_ref, b4_ref, w5_ref, b5_ref, o_ref)


def _out_stage(h_chunk, b3, w4, b4, w5two, b5two, obuf, chunk):
    off = chunk * CNB
    specs = [
        pl.BlockSpec((MT, D), lambda i: (i, 0)),
        pl.BlockSpec((1, D), lambda i: (0, 0)),
        pl.BlockSpec((D, D), lambda i: (0, 0)),
        pl.BlockSpec((1, D), lambda i: (0, 0)),
        pl.BlockSpec((D, 2), lambda i: (0, 0)),
        pl.BlockSpec((1, 2), lambda i: (0, 0)),
    ]
    args = [h_chunk, b3, w4, b4, w5two, b5two]
    body = _out_body
    aliases = {}
    if obuf is not None:
        specs.append(pl.BlockSpec(memory_space=pl.ANY))
        args.append(obuf)
        body = _out_body_alias
        aliases = {6: 0}
    return pl.pallas_call(
        body,
        grid=(CNB,),
        in_specs=specs,
        out_specs=pl.BlockSpec((MT, 2), lambda i: (off + i, 0)),
        out_shape=jax.ShapeDtypeStruct((N_NODES - 1, 2), jnp.float32),
        input_output_aliases=aliases,
    )(*args)


def _sc_mesh():
    return plsc.VectorSubcoreMesh(core_axis_name="c", subcore_axis_name="s")


def _accum_pair(acc, ga, gb):
    @pl.loop(0, B, step=2)
    def _row(r):
        for rr in range(2):
            for j in range(D // 16):
                sl = (pl.ds(r + rr, 1), pl.ds(j * 16, 16))
                plsc.addupdate(acc.at[sl], ga[sl] + gb[sl])


def _gather_sum(table, idx_chunk):
    per_w = PER_W // CHUNKS

    @functools.partial(
        pl.kernel,
        out_type=jax.ShapeDtypeStruct((CROWS, D), jnp.float32),
        mesh=_sc_mesh(),
        scratch_types=[
            pltpu.VMEM((NEIGH, B), jnp.int32),
            pltpu.VMEM((NEIGH, B), jnp.int32),
            pltpu.VMEM((B, D), jnp.float32),
            pltpu.VMEM((B, D), jnp.float32),
            pltpu.VMEM((B, D), jnp.float32),
            pltpu.VMEM((B, D), jnp.float32),
            pltpu.VMEM((B, D), jnp.float32),
            pltpu.VMEM((B, D), jnp.float32),
            pltpu.SemaphoreType.DMA,
            pltpu.SemaphoreType.DMA,
            pltpu.SemaphoreType.DMA,
            pltpu.SemaphoreType.DMA,
            pltpu.SemaphoreType.DMA,
            pltpu.SemaphoreType.DMA,
            pltpu.SemaphoreType.DMA,
            pltpu.SemaphoreType.DMA,
            pltpu.SemaphoreType.DMA,
            pltpu.SemaphoreType.DMA,
        ],
    )
    def k(table_hbm, idx_hbm, out_hbm, ix0, ix1, ac0, ac1, g0, g1, g2, g3,
          si0, si1, sa0, sa1, s0, s1, s2, s3, sw0, sw1):
        cid = lax.axis_index("c")
        sid = lax.axis_index("s")
        wid = sid * 2 + cid

        ix = [ix0, ix1]
        si = [si0, si1]
        ac = [ac0, ac1]
        sa = [sa0, sa1]
        sw = [sw0, sw1]
        gbuf = [g0, g1, g2, g3]
        gs = [s0, s1, s2, s3]

        def gath(ib, kk, buf, sem):
            return pltpu.async_copy(table_hbm.at[ib.at[kk]], buf, sem)

        def rot(b):
            return (0, 1, 2, 3) if b % 2 == 0 else (2, 3, 0, 1)

        pltpu.sync_copy(idx_hbm.at[wid * NBLK_C], ix0)
        pend = {}
        pend[(0, 0)] = gath(ix0, 0, ac0, sa0)
        r = rot(0)
        for kk in range(1, 5):
            pend[(0, kk)] = gath(ix0, kk, gbuf[r[kk - 1]], gs[r[kk - 1]])
        ixp = [None, None]
        if NBLK_C > 1:
            ixp[1] = pltpu.async_copy(idx_hbm.at[wid * NBLK_C + 1], ix1, si1)
        wbh = [None, None]

        for b in range(NBLK_C):
            p = b % 2
            q = (b + 1) % 2
            r = rot(b)
            rn = rot(b + 1)
            base = wid * per_w + b * B
            pend.pop((b, 0)).wait()
            pend.pop((b, 1)).wait()
            pend.pop((b, 2)).wait()
            _accum_pair(ac[p], gbuf[r[0]], gbuf[r[1]])
            pend[(b, 5)] = gath(ix[p], 5, gbuf[r[0]], gs[r[0]])
            pend[(b, 6)] = gath(ix[p], 6, gbuf[r[1]], gs[r[1]])
            pend.pop((b, 3)).wait()
            pend.pop((b, 4)).wait()
            _accum_pair(ac[p], gbuf[r[2]], gbuf[r[3]])
            if b + 1 < NBLK_C:
                ixp[q].wait()
                if wbh[q] is not None:
                    wbh[q].wait()
                pend[(b + 1, 0)] = gath(ix[q], 0, ac[q], sa[q])
                pend[(b + 1, 1)] = gath(ix[q], 1, gbuf[rn[0]], gs[rn[0]])
                pend[(b + 1, 2)] = gath(ix[q], 2, gbuf[rn[1]], gs[rn[1]])
            pend.pop((b, 5)).wait()
            pend.pop((b, 6)).wait()
            _accum_pair(ac[p], gbuf[r[0]], gbuf[r[1]])
            wbh[p] = pltpu.async_copy(ac[p], out_hbm.at[pl.ds(base, B)],
                                      sw[p])
            if b + 1 < NBLK_C:
                pend[(b + 1, 3)] = gath(ix[q], 3, gbuf[rn[2]], gs[rn[2]])
                pend[(b + 1, 4)] = gath(ix[q], 4, gbuf[rn[3]], gs[rn[3]])
                if b + 2 < NBLK_C:
                    ixp[p] = pltpu.async_copy(
                        idx_hbm.at[wid * NBLK_C + b + 2], ix[p], si[p])

        for h in wbh:
            if h is not None:
                h.wait()

    return k(table, idx_chunk)


def kernel(logits, G, W1, b1, W2, b2, W3, b3, W4, b4, W5, b5):
    f32 = jnp.float32
    bf16 = jnp.bfloat16
    x = logits.astype(f32)
    npadrows = NPAD - N_NODES

    Gi = G.astype(jnp.int32)
    gpad = (jnp.arange(npadrows, dtype=jnp.int32)[:, None] * NEIGH
            + jnp.arange(NEIGH, dtype=jnp.int32)[None, :]) % N_NODES
    Gp = jnp.concatenate([Gi, gpad], axis=0)
    idxT = (Gp.T + (jnp.arange(NEIGH, dtype=jnp.int32) * NPAD)[:, None])
    idxT = idxT.astype(jnp.int32)
    idx_chunks = (idxT.reshape(NEIGH, CHUNKS, NW, NBLK_C, B)
                  .transpose(1, 2, 3, 0, 4)
                  .reshape(CHUNKS, NW * NBLK_C, NEIGH, B))

    w1s = W1.reshape(NEIGH, D, D).transpose(1, 0, 2).reshape(D, NEIGH * D)
    w1s = w1s.astype(bf16)
    w3s = W3.reshape(NEIGH, D, D).transpose(1, 0, 2).reshape(D, NEIGH * D)
    w3s = w3s.astype(bf16)
    b1r = b1.astype(f32).reshape(1, D)
    b2r = b2.astype(f32).reshape(1, D)
    b3r = b3.astype(f32).reshape(1, D)
    b4r = b4.astype(f32).reshape(1, D)
    w2c = W2.astype(bf16)
    w4c = W4.astype(bf16)
    wd = jnp.stack([W5[:, 0] - W5[:, 1], W5[:, 1] - W5[:, 0]], axis=1)
    w5two = wd.astype(bf16)
    bd = jnp.stack([b5[0] - b5[1], b5[1] - b5[0]])
    b5two = bd.astype(f32).reshape(1, 2)

    y1 = _mm7(x, w1s).reshape(NEIGH * NPAD, D)

    h1 = [_gather_sum(y1, idx_chunks[c]) for c in range(CHUNKS)]

    ybuf = None
    for c in range(CHUNKS):
        ybuf = _mid(h1[c], b1r, w2c, b2r, w3s, ybuf, c)
    y2 = ybuf.reshape(NEIGH * NPAD, D)

    h2 = [_gather_sum(y2, idx_chunks[c]) for c in range(CHUNKS)]

    obuf = None
    for c in range(CHUNKS):
        obuf = _out_stage(h2[c], b3r, w4c, b4r, w5two, b5two, obuf, c)
    return obuf
